# baseline (speedup 1.0000x reference)
import functools

import jax
import jax.numpy as jnp
from jax import lax
from jax.experimental import pallas as pl
from jax.experimental.pallas import tpu as pltpu
from jax.experimental.pallas import tpu_sc as plsc

N = 100000
E = 3200000
G = 1024
K = 16

NUM_CORES = 2
NUM_SUBCORES = 16
NW = NUM_CORES * NUM_SUBCORES

CH = 128
GC = 2
GCH = GC * CH
GPT = 392
E_PAD = NW * GPT * GC * CH
NP_PAD = 102400
DUMMY = NP_PAD
N_ACC = NP_PAD + 128
ZROWS = N_ACC // NUM_SUBCORES
CHT_P = NP_PAD // NW // CH
G_ACC = 1152
GZ = G_ACC // NUM_SUBCORES


def _mesh():
    return plsc.VectorSubcoreMesh(core_axis_name="c", subcore_axis_name="s")


def _edge_pass(U, V, sdp, eap, wd2):
    @functools.partial(
        pl.kernel,
        mesh=_mesh(),
        compiler_params=pltpu.CompilerParams(use_tc_tiling_on_sc=False),
        out_type=jax.ShapeDtypeStruct((NUM_CORES, NP_PAD, K), jnp.float32),
        scratch_types=[
            pltpu.VMEM((3, 2, GCH), jnp.int32),
            pltpu.VMEM((3, GCH), jnp.float32),
            pltpu.VMEM((2, GCH, K), jnp.float32),
            pltpu.VMEM((2, GCH, K), jnp.float32),
            pltpu.VMEM((2, GCH, K), jnp.float32),
            pltpu.VMEM((K,), jnp.float32),
            pltpu.VMEM_SHARED((N_ACC, K), jnp.float32),
            pltpu.SemaphoreType.DMA,
            pltpu.SemaphoreType.DMA,
            pltpu.SemaphoreType.DMA,
        ],
    )
    def k(u_hbm, v_hbm, sd_hbm, ea_hbm, wd_hbm, out_hbm,
          sdg, eag, ub, vb, sb, wdb, acc, sem_ig, sem_ga, sem_sc):
        cid = lax.axis_index("c")
        sid = lax.axis_index("s")
        wid = cid * NUM_SUBCORES + sid

        def zrow(i, _):
            ub[0, i, :] = jnp.zeros((K,), jnp.float32)
            return 0
        lax.fori_loop(0, GCH, zrow, 0)
        base = sid * ZROWS

        def zchunk(j, _):
            pltpu.sync_copy(ub.at[0], acc.at[pl.ds(base + j * GCH, GCH)])
            return 0
        lax.fori_loop(0, ZROWS // GCH, zchunk, 0)
        pltpu.sync_copy(ub.at[0, pl.ds(0, ZROWS % GCH)],
                        acc.at[pl.ds(base + (ZROWS // GCH) * GCH, ZROWS % GCH)])
        plsc.subcore_barrier()

        pltpu.sync_copy(wd_hbm, wdb)
        wd2v = wdb[:]
        one = jnp.float32(1.0)
        two = jnp.float32(2.0)
        lanes = [jnp.full((K, 1), i, jnp.int32) for i in range(K)]
        _dnums = lax.GatherDimensionNumbers(
            offset_dims=(), collapsed_slice_dims=(0,), start_index_map=(0,))

        def _bcast(av, i):
            return lax.gather(av, lanes[i], _dnums, (1,),
                              mode=lax.GatherScatterMode.PROMISE_IN_BOUNDS)

        def idx_issue(g):
            gg = wid * GPT + g
            s = lax.rem(g, 3)
            pltpu.async_copy(sd_hbm.at[gg], sdg.at[s], sem_ig)
            pltpu.async_copy(ea_hbm.at[gg], eag.at[s], sem_ig)

        def idx_wait(g):
            gg = wid * GPT + g
            s = lax.rem(g, 3)
            pltpu.make_async_copy(sd_hbm.at[gg], sdg.at[s], sem_ig).wait()
            pltpu.make_async_copy(ea_hbm.at[gg], eag.at[s], sem_ig).wait()

        def gath_issue(g, p):
            s = lax.rem(g, 3)
            pltpu.async_copy(u_hbm.at[sdg.at[s, 0]], ub.at[p], sem_ga)
            pltpu.async_copy(v_hbm.at[sdg.at[s, 1]], vb.at[p], sem_ga)

        def gath_wait(g, p):
            s = lax.rem(g, 3)
            pltpu.make_async_copy(
                u_hbm.at[sdg.at[s, 0]], ub.at[p], sem_ga).wait()
            pltpu.make_async_copy(
                v_hbm.at[sdg.at[s, 1]], vb.at[p], sem_ga).wait()

        def scat_issue(g, p):
            s = lax.rem(g, 3)
            pltpu.async_copy(sb.at[p], acc.at[sdg.at[s, 1]], sem_sc, add=True)

        def scat_wait(g, p):
            s = lax.rem(g, 3)
            pltpu.make_async_copy(sb.at[p], acc.at[sdg.at[s, 1]], sem_sc).wait()

        def compute(g, p):
            s = lax.rem(g, 3)

            def qblk(q, _):
                av = eag[s, pl.ds(q * K, K)]
                for i in range(K):
                    e = q * K + i
                    ai = _bcast(av, i)
                    z = ub[p, e, :] + vb[p, e, :] + ai * wd2v
                    sb[p, e, :] = one - two / (jnp.exp(z) + one)
                return 0
            lax.fori_loop(0, GCH // K, qblk, 0)

        def step(g, p):
            @pl.when(g > 0)
            def _():
                scat_wait(g - 1, 1 - p)

            @pl.when(g < GPT - 1)
            def _():
                idx_wait(g + 1)

            @pl.when(g < GPT - 2)
            def _():
                idx_issue(g + 2)
            gath_wait(g, p)

            @pl.when(g < GPT - 1)
            def _():
                gath_issue(g + 1, 1 - p)
            compute(g, p)
            scat_issue(g, p)

        idx_issue(0)
        idx_issue(1)
        idx_wait(0)
        gath_issue(0, 0)

        def body(t, _):
            step(2 * t, 0)
            step(2 * t + 1, 1)
            return 0
        lax.fori_loop(0, GPT // 2, body, 0)
        scat_wait(GPT - 1, 1)
        plsc.subcore_barrier()

        @pl.when(sid == 0)
        def _():
            pltpu.sync_copy(acc.at[pl.ds(0, NP_PAD)], out_hbm.at[cid])

    return k(U, V, sdp, eap, wd2)


def _pool(parts, batchp):
    @functools.partial(
        pl.kernel,
        mesh=_mesh(),
        compiler_params=pltpu.CompilerParams(use_tc_tiling_on_sc=False),
        out_type=(
            jax.ShapeDtypeStruct((NUM_CORES, G, K), jnp.float32),
            jax.ShapeDtypeStruct((NUM_CORES, G, K), jnp.float32),
        ),
        scratch_types=[
            pltpu.VMEM((CH,), jnp.int32),
            pltpu.VMEM((CH, K), jnp.float32),
            pltpu.VMEM((CH, K), jnp.float32),
            pltpu.VMEM((CH, K), jnp.float32),
            pltpu.VMEM_SHARED((G_ACC, K), jnp.float32),
            pltpu.VMEM_SHARED((G_ACC, K), jnp.float32),
        ],
    )
    def k(parts_hbm, batch_hbm, rsum_hbm, cnt_hbm,
          bb, h0, h1, onesb, rs, cs):
        cid = lax.axis_index("c")
        sid = lax.axis_index("s")
        wid = cid * NUM_SUBCORES + sid

        def fill(i, _):
            h0[i, :] = jnp.zeros((K,), jnp.float32)
            onesb[i, :] = jnp.ones((K,), jnp.float32)
            return 0
        lax.fori_loop(0, CH, fill, 0)
        base = sid * GZ
        pltpu.sync_copy(h0.at[pl.ds(0, GZ)], rs.at[pl.ds(base, GZ)])
        pltpu.sync_copy(h0.at[pl.ds(0, GZ)], cs.at[pl.ds(base, GZ)])
        plsc.subcore_barrier()

        def chunk(ch, _):
            g = wid * CHT_P + ch
            pltpu.sync_copy(batch_hbm.at[g], bb)
            pltpu.sync_copy(parts_hbm.at[0, pl.ds(g * CH, CH)], h0)
            pltpu.sync_copy(parts_hbm.at[1, pl.ds(g * CH, CH)], h1)

            def row(e, _):
                h0[e, :] = h0[e, :] + h1[e, :]
                return 0
            lax.fori_loop(0, CH, row, 0)
            pltpu.sync_copy(h0, rs.at[bb], add=True)
            pltpu.sync_copy(onesb, cs.at[bb], add=True)
            return 0
        lax.fori_loop(0, CHT_P, chunk, 0)
        plsc.subcore_barrier()

        @pl.when(sid == 0)
        def _():
            pltpu.sync_copy(rs.at[pl.ds(0, G)], rsum_hbm.at[cid])
            pltpu.sync_copy(cs.at[pl.ds(0, G)], cnt_hbm.at[cid])

    return k(parts, batchp)


_R = 1000


def _proj0(xp, Wu, Wv, bv):
    def body(xp_ref, wu_ref, wv_ref, bv_ref, u_ref, v_ref):
        xpb = xp_ref[...]
        u_ref[...] = jnp.dot(xpb, wu_ref[...], preferred_element_type=jnp.float32)
        v_ref[...] = (jnp.dot(xpb, wv_ref[...], preferred_element_type=jnp.float32)
                      + bv_ref[...])

    return pl.pallas_call(
        body,
        grid=(N // _R,),
        in_specs=[
            pl.BlockSpec((_R, 8), lambda i: (i, 0)),
            pl.BlockSpec((8, K), lambda i: (0, 0)),
            pl.BlockSpec((8, K), lambda i: (0, 0)),
            pl.BlockSpec((1, K), lambda i: (0, 0)),
        ],
        out_specs=[
            pl.BlockSpec((_R, K), lambda i: (i, 0)),
            pl.BlockSpec((_R, K), lambda i: (i, 0)),
        ],
        out_shape=[
            jax.ShapeDtypeStruct((N, K), jnp.float32),
            jax.ShapeDtypeStruct((N, K), jnp.float32),
        ],
    )(xp, Wu, Wv, bv)


def _proj1(parts, p, Wua, Wuc, Wva, Wvc, bv):
    def body(pa_ref, pb_ref, p_ref, wua_ref, wuc_ref, wva_ref, wvc_ref,
             bv_ref, u_ref, v_ref):
        h = pa_ref[0] + pb_ref[0]
        pb = p_ref[...]
        u_ref[...] = (jnp.dot(h, wua_ref[...], preferred_element_type=jnp.float32)
                      + jnp.dot(pb, wuc_ref[...], preferred_element_type=jnp.float32))
        v_ref[...] = (jnp.dot(h, wva_ref[...], preferred_element_type=jnp.float32)
                      + jnp.dot(pb, wvc_ref[...], preferred_element_type=jnp.float32)
                      + bv_ref[...])

    return pl.pallas_call(
        body,
        grid=(N // _R,),
        in_specs=[
            pl.BlockSpec((1, _R, K), lambda i: (0, i, 0)),
            pl.BlockSpec((1, _R, K), lambda i: (1, i, 0)),
            pl.BlockSpec((_R, 3), lambda i: (i, 0)),
            pl.BlockSpec((K, K), lambda i: (0, 0)),
            pl.BlockSpec((3, K), lambda i: (0, 0)),
            pl.BlockSpec((K, K), lambda i: (0, 0)),
            pl.BlockSpec((3, K), lambda i: (0, 0)),
            pl.BlockSpec((1, K), lambda i: (0, 0)),
        ],
        out_specs=[
            pl.BlockSpec((_R, K), lambda i: (i, 0)),
            pl.BlockSpec((_R, K), lambda i: (i, 0)),
        ],
        out_shape=[
            jax.ShapeDtypeStruct((N, K), jnp.float32),
            jax.ShapeDtypeStruct((N, K), jnp.float32),
        ],
    )(parts, parts, p, Wua, Wuc, Wva, Wvc, bv)


def _final(rsum, cnt, Wp, bp):
    def body(rs_ref, cn_ref, wp_ref, bp_ref, pred_ref, rep_ref):
        rs = rs_ref[0] + rs_ref[1]
        c = cn_ref[0, :, 0:1] + cn_ref[1, :, 0:1]
        rep = rs / jnp.maximum(c, 1.0)
        rep_ref[...] = rep
        pred_ref[...] = (jnp.dot(rep, wp_ref[...], preferred_element_type=jnp.float32)
                         + bp_ref[...])

    return pl.pallas_call(
        body,
        out_shape=[
            jax.ShapeDtypeStruct((G, 1), jnp.float32),
            jax.ShapeDtypeStruct((G, K), jnp.float32),
        ],
    )(rsum, cnt, Wp, bp)


def kernel(x, p, edge_index, edge_attr, batch, W0, b0, W1, b1, Wp, bp):
    src = edge_index[0]
    dst = edge_index[1]
    pad = E_PAD - E
    srcp = jnp.concatenate([src, jnp.zeros((pad,), jnp.int32)]
                           ).reshape(-1, GCH)
    dstp = jnp.concatenate([dst, jnp.full((pad,), DUMMY, jnp.int32)]
                           ).reshape(-1, GCH)
    sdp = jnp.stack([srcp, dstp], axis=1)
    eap = jnp.concatenate([edge_attr[:, 0], jnp.zeros((pad,), jnp.float32)]
                          ).reshape(-1, GCH)
    batchp = jnp.concatenate([batch, jnp.full((NP_PAD - N,), G, jnp.int32)]
                             ).reshape(-1, CH)

    Wa0, Wb0, Wc0, wd0 = W0[0:5], W0[5:10], W0[10:13], W0[13]
    Wu0 = 2.0 * jnp.concatenate([Wa0, -Wc0], axis=0)
    Wv0 = 2.0 * jnp.concatenate([Wb0, Wc0], axis=0)
    bv0 = (2.0 * b0).reshape(1, K)
    wd20 = 2.0 * wd0
    xp = jnp.concatenate([x, p], axis=1)

    U0, V0 = _proj0(xp, Wu0, Wv0, bv0)
    parts0 = _edge_pass(U0, V0, sdp, eap, wd20)

    Wa1, Wb1, Wc1, wd1 = W1[0:16], W1[16:32], W1[32:35], W1[35]
    U1, V1 = _proj1(parts0, p, 2.0 * Wa1, -2.0 * Wc1, 2.0 * Wb1, 2.0 * Wc1,
                    (2.0 * b1).reshape(1, K))
    parts1 = _edge_pass(U1, V1, sdp, eap, 2.0 * wd1)

    rsum, cnt = _pool(parts1, batchp)
    pred, rep = _final(rsum, cnt, Wp, bp.reshape(1, 1))
    return (pred, rep)


# speedup vs baseline: 1.0832x; 1.0832x over previous
import functools

import jax
import jax.numpy as jnp
from jax import lax
from jax.experimental import pallas as pl
from jax.experimental.pallas import tpu as pltpu
from jax.experimental.pallas import tpu_sc as plsc

N = 100000
E = 3200000
G = 1024
K = 16

NUM_CORES = 2
NUM_SUBCORES = 16
NW = NUM_CORES * NUM_SUBCORES

GCH = 200
GPT = 500
EPT = GPT * GCH
N_ACC = N
ZROWS = N_ACC // NUM_SUBCORES
PCH = 125
PCT = 25
G_ACC = G
GZ = G_ACC // NUM_SUBCORES


def _mesh():
    return plsc.VectorSubcoreMesh(core_axis_name="c", subcore_axis_name="s")


def _edge_pass(U, V, ei, eaf, wd2):
    @functools.partial(
        pl.kernel,
        mesh=_mesh(),
        compiler_params=pltpu.CompilerParams(use_tc_tiling_on_sc=False),
        out_type=jax.ShapeDtypeStruct((NUM_CORES, N, K), jnp.float32),
        scratch_types=[
            pltpu.VMEM((3, GCH), jnp.int32),
            pltpu.VMEM((3, GCH), jnp.int32),
            pltpu.VMEM((3, GCH), jnp.float32),
            pltpu.VMEM((2, GCH, K), jnp.float32),
            pltpu.VMEM((2, GCH, K), jnp.float32),
            pltpu.VMEM((2, GCH, K), jnp.float32),
            pltpu.VMEM((K,), jnp.float32),
            pltpu.VMEM_SHARED((N_ACC, K), jnp.float32),
            pltpu.SemaphoreType.DMA,
            pltpu.SemaphoreType.DMA,
            pltpu.SemaphoreType.DMA,
        ],
    )
    def k(u_hbm, v_hbm, ei_hbm, ea_hbm, wd_hbm, out_hbm,
          srcg, dstg, eag, ub, vb, sb, wdb, acc, sem_ig, sem_ga, sem_sc):
        cid = lax.axis_index("c")
        sid = lax.axis_index("s")
        wid = cid * NUM_SUBCORES + sid

        def zrow(i, _):
            ub[0, i, :] = jnp.zeros((K,), jnp.float32)
            return 0
        lax.fori_loop(0, GCH, zrow, 0)
        zbase = sid * ZROWS

        def zchunk(j, _):
            pltpu.sync_copy(ub.at[0], acc.at[pl.ds(zbase + j * GCH, GCH)])
            return 0
        lax.fori_loop(0, ZROWS // GCH, zchunk, 0)
        pltpu.sync_copy(ub.at[0, pl.ds(0, ZROWS % GCH)],
                        acc.at[pl.ds(zbase + (ZROWS // GCH) * GCH, ZROWS % GCH)])
        plsc.subcore_barrier()

        pltpu.sync_copy(wd_hbm, wdb)
        wd2v = wdb[:]
        one = jnp.float32(1.0)
        two = jnp.float32(2.0)
        lanes = [jnp.full((K, 1), i, jnp.int32) for i in range(K)]
        _dnums = lax.GatherDimensionNumbers(
            offset_dims=(), collapsed_slice_dims=(0,), start_index_map=(0,))

        def _bcast(av, i):
            return lax.gather(av, lanes[i], _dnums, (1,),
                              mode=lax.GatherScatterMode.PROMISE_IN_BOUNDS)

        def idx_issue(g):
            b = wid * EPT + g * GCH
            s = lax.rem(g, 3)
            pltpu.async_copy(ei_hbm.at[0, pl.ds(b, GCH)], srcg.at[s], sem_ig)
            pltpu.async_copy(ei_hbm.at[1, pl.ds(b, GCH)], dstg.at[s], sem_ig)
            pltpu.async_copy(ea_hbm.at[pl.ds(b, GCH)], eag.at[s], sem_ig)

        def idx_wait(g):
            b = wid * EPT + g * GCH
            s = lax.rem(g, 3)
            pltpu.make_async_copy(
                ei_hbm.at[0, pl.ds(b, GCH)], srcg.at[s], sem_ig).wait()
            pltpu.make_async_copy(
                ei_hbm.at[1, pl.ds(b, GCH)], dstg.at[s], sem_ig).wait()
            pltpu.make_async_copy(
                ea_hbm.at[pl.ds(b, GCH)], eag.at[s], sem_ig).wait()

        def gath_issue(g, p):
            s = lax.rem(g, 3)
            pltpu.async_copy(u_hbm.at[srcg.at[s]], ub.at[p], sem_ga)
            pltpu.async_copy(v_hbm.at[dstg.at[s]], vb.at[p], sem_ga)

        def gath_wait(g, p):
            s = lax.rem(g, 3)
            pltpu.make_async_copy(u_hbm.at[srcg.at[s]], ub.at[p], sem_ga).wait()
            pltpu.make_async_copy(v_hbm.at[dstg.at[s]], vb.at[p], sem_ga).wait()

        def scat_issue(g, p):
            s = lax.rem(g, 3)
            pltpu.async_copy(sb.at[p], acc.at[dstg.at[s]], sem_sc, add=True)

        def scat_wait(g, p):
            s = lax.rem(g, 3)
            pltpu.make_async_copy(sb.at[p], acc.at[dstg.at[s]], sem_sc).wait()

        def edge16(p, av, base_e, lo):
            for i in range(lo, K):
                e = base_e + i
                ai = _bcast(av, i)
                z = ub[p, e, :] + vb[p, e, :] + ai * wd2v
                sb[p, e, :] = one - two / (jnp.exp(z) + one)

        def compute(g, p):
            s = lax.rem(g, 3)

            def qblk(q, _):
                av = eag[s, pl.ds(q * K, K)]
                edge16(p, av, q * K, 0)
                return 0
            lax.fori_loop(0, GCH // K, qblk, 0)
            av = eag[s, pl.ds(GCH - K, K)]
            edge16(p, av, GCH - K, K - (GCH - (GCH // K) * K))

        def step(g, p):
            @pl.when(g > 0)
            def _():
                scat_wait(g - 1, 1 - p)

            @pl.when(g < GPT - 1)
            def _():
                idx_wait(g + 1)

            @pl.when(g < GPT - 2)
            def _():
                idx_issue(g + 2)
            gath_wait(g, p)

            @pl.when(g < GPT - 1)
            def _():
                gath_issue(g + 1, 1 - p)
            compute(g, p)
            scat_issue(g, p)

        idx_issue(0)
        idx_issue(1)
        idx_wait(0)
        gath_issue(0, 0)

        def body(t, _):
            step(2 * t, 0)
            step(2 * t + 1, 1)
            return 0
        lax.fori_loop(0, GPT // 2, body, 0)
        scat_wait(GPT - 1, 1)
        plsc.subcore_barrier()

        @pl.when(sid == 0)
        def _():
            pltpu.sync_copy(acc, out_hbm.at[cid])

    return k(U, V, ei, eaf, wd2)


def _pool(parts, batchp):
    @functools.partial(
        pl.kernel,
        mesh=_mesh(),
        compiler_params=pltpu.CompilerParams(use_tc_tiling_on_sc=False),
        out_type=(
            jax.ShapeDtypeStruct((NUM_CORES, G, K), jnp.float32),
            jax.ShapeDtypeStruct((NUM_CORES, G, K), jnp.float32),
        ),
        scratch_types=[
            pltpu.VMEM((PCH,), jnp.int32),
            pltpu.VMEM((PCH, K), jnp.float32),
            pltpu.VMEM((PCH, K), jnp.float32),
            pltpu.VMEM((PCH, K), jnp.float32),
            pltpu.VMEM_SHARED((G_ACC, K), jnp.float32),
            pltpu.VMEM_SHARED((G_ACC, K), jnp.float32),
        ],
    )
    def k(parts_hbm, batch_hbm, rsum_hbm, cnt_hbm,
          bb, h0, h1, onesb, rs, cs):
        cid = lax.axis_index("c")
        sid = lax.axis_index("s")
        wid = cid * NUM_SUBCORES + sid

        def fill(i, _):
            h0[i, :] = jnp.zeros((K,), jnp.float32)
            onesb[i, :] = jnp.ones((K,), jnp.float32)
            return 0
        lax.fori_loop(0, PCH, fill, 0)
        zb = sid * GZ
        pltpu.sync_copy(h0.at[pl.ds(0, GZ)], rs.at[pl.ds(zb, GZ)])
        pltpu.sync_copy(h0.at[pl.ds(0, GZ)], cs.at[pl.ds(zb, GZ)])
        plsc.subcore_barrier()

        def chunk(ch, _):
            g = wid * PCT + ch
            pltpu.sync_copy(batch_hbm.at[g], bb)
            pltpu.sync_copy(parts_hbm.at[0, pl.ds(g * PCH, PCH)], h0)
            pltpu.sync_copy(parts_hbm.at[1, pl.ds(g * PCH, PCH)], h1)
            pltpu.sync_copy(h0, rs.at[bb], add=True)
            pltpu.sync_copy(h1, rs.at[bb], add=True)
            pltpu.sync_copy(onesb, cs.at[bb], add=True)
            return 0
        lax.fori_loop(0, PCT, chunk, 0)
        plsc.subcore_barrier()

        @pl.when(sid == 0)
        def _():
            pltpu.sync_copy(rs, rsum_hbm.at[cid])
            pltpu.sync_copy(cs, cnt_hbm.at[cid])

    return k(parts, batchp)


_R = 5000


def _proj0(xp, Wu, Wv, bv):
    def body(xp_ref, wu_ref, wv_ref, bv_ref, u_ref, v_ref):
        xpb = xp_ref[...]
        u_ref[...] = jnp.dot(xpb, wu_ref[...], preferred_element_type=jnp.float32)
        v_ref[...] = (jnp.dot(xpb, wv_ref[...], preferred_element_type=jnp.float32)
                      + bv_ref[...])

    return pl.pallas_call(
        body,
        grid=(N // _R,),
        in_specs=[
            pl.BlockSpec((_R, 8), lambda i: (i, 0)),
            pl.BlockSpec((8, K), lambda i: (0, 0)),
            pl.BlockSpec((8, K), lambda i: (0, 0)),
            pl.BlockSpec((1, K), lambda i: (0, 0)),
        ],
        out_specs=[
            pl.BlockSpec((_R, K), lambda i: (i, 0)),
            pl.BlockSpec((_R, K), lambda i: (i, 0)),
        ],
        out_shape=[
            jax.ShapeDtypeStruct((N, K), jnp.float32),
            jax.ShapeDtypeStruct((N, K), jnp.float32),
        ],
    )(xp, Wu, Wv, bv)


def _proj1(parts, p, Wua, Wuc, Wva, Wvc, bv):
    def body(pa_ref, pb_ref, p_ref, wua_ref, wuc_ref, wva_ref, wvc_ref,
             bv_ref, u_ref, v_ref):
        h = pa_ref[0] + pb_ref[0]
        pb = p_ref[...]
        u_ref[...] = (jnp.dot(h, wua_ref[...], preferred_element_type=jnp.float32)
                      + jnp.dot(pb, wuc_ref[...], preferred_element_type=jnp.float32))
        v_ref[...] = (jnp.dot(h, wva_ref[...], preferred_element_type=jnp.float32)
                      + jnp.dot(pb, wvc_ref[...], preferred_element_type=jnp.float32)
                      + bv_ref[...])

    return pl.pallas_call(
        body,
        grid=(N // _R,),
        in_specs=[
            pl.BlockSpec((1, _R, K), lambda i: (0, i, 0)),
            pl.BlockSpec((1, _R, K), lambda i: (1, i, 0)),
            pl.BlockSpec((_R, 3), lambda i: (i, 0)),
            pl.BlockSpec((K, K), lambda i: (0, 0)),
            pl.BlockSpec((3, K), lambda i: (0, 0)),
            pl.BlockSpec((K, K), lambda i: (0, 0)),
            pl.BlockSpec((3, K), lambda i: (0, 0)),
            pl.BlockSpec((1, K), lambda i: (0, 0)),
        ],
        out_specs=[
            pl.BlockSpec((_R, K), lambda i: (i, 0)),
            pl.BlockSpec((_R, K), lambda i: (i, 0)),
        ],
        out_shape=[
            jax.ShapeDtypeStruct((N, K), jnp.float32),
            jax.ShapeDtypeStruct((N, K), jnp.float32),
        ],
    )(parts, parts, p, Wua, Wuc, Wva, Wvc, bv)


def _final(rsum, cnt, Wp, bp):
    def body(rs_ref, cn_ref, wp_ref, bp_ref, pred_ref, rep_ref):
        rs = rs_ref[0] + rs_ref[1]
        c = cn_ref[0, :, 0:1] + cn_ref[1, :, 0:1]
        rep = rs / jnp.maximum(c, 1.0)
        rep_ref[...] = rep
        pred_ref[...] = (jnp.dot(rep, wp_ref[...], preferred_element_type=jnp.float32)
                         + bp_ref[...])

    return pl.pallas_call(
        body,
        out_shape=[
            jax.ShapeDtypeStruct((G, 1), jnp.float32),
            jax.ShapeDtypeStruct((G, K), jnp.float32),
        ],
    )(rsum, cnt, Wp, bp)


def kernel(x, p, edge_index, edge_attr, batch, W0, b0, W1, b1, Wp, bp):
    eaf = edge_attr.reshape(E)
    batchp = batch.reshape(-1, PCH)

    Wa0, Wb0, Wc0, wd0 = W0[0:5], W0[5:10], W0[10:13], W0[13]
    Wu0 = 2.0 * jnp.concatenate([Wa0, -Wc0], axis=0)
    Wv0 = 2.0 * jnp.concatenate([Wb0, Wc0], axis=0)
    bv0 = (2.0 * b0).reshape(1, K)
    wd20 = 2.0 * wd0
    xp = jnp.concatenate([x, p], axis=1)

    U0, V0 = _proj0(xp, Wu0, Wv0, bv0)
    parts0 = _edge_pass(U0, V0, edge_index, eaf, wd20)

    Wa1, Wb1, Wc1, wd1 = W1[0:16], W1[16:32], W1[32:35], W1[35]
    U1, V1 = _proj1(parts0, p, 2.0 * Wa1, -2.0 * Wc1, 2.0 * Wb1, 2.0 * Wc1,
                    (2.0 * b1).reshape(1, K))
    parts1 = _edge_pass(U1, V1, edge_index, eaf, 2.0 * wd1)

    rsum, cnt = _pool(parts1, batchp)
    pred, rep = _final(rsum, cnt, Wp, bp.reshape(1, 1))
    return (pred, rep)


# speedup vs baseline: 1.1869x; 1.0957x over previous
import functools

import jax
import jax.numpy as jnp
from jax import lax
from jax.experimental import pallas as pl
from jax.experimental.pallas import tpu as pltpu
from jax.experimental.pallas import tpu_sc as plsc

N = 100000
E = 3200000
G = 1024
K = 16

NUM_CORES = 2
NUM_SUBCORES = 16
NW = NUM_CORES * NUM_SUBCORES

GCH = 200
GPT = 500
EPT = GPT * GCH
N_ACC = N
ZROWS = N_ACC // NUM_SUBCORES
PCH = 125
PCT = 25
G_ACC = G
GZ = G_ACC // NUM_SUBCORES


def _mesh():
    return plsc.VectorSubcoreMesh(core_axis_name="c", subcore_axis_name="s")


def _edge_pass(U, V, ei, eaf, wd2):
    @functools.partial(
        pl.kernel,
        mesh=_mesh(),
        compiler_params=pltpu.CompilerParams(use_tc_tiling_on_sc=False),
        out_type=jax.ShapeDtypeStruct((NUM_CORES, N, K), jnp.float32),
        scratch_types=[
            pltpu.VMEM((3, GCH), jnp.int32),
            pltpu.VMEM((3, GCH), jnp.int32),
            pltpu.VMEM((3, GCH), jnp.float32),
            pltpu.VMEM((2, GCH, K), jnp.float32),
            pltpu.VMEM((2, GCH, K), jnp.float32),
            pltpu.VMEM((2, GCH, K), jnp.float32),
            pltpu.VMEM((K,), jnp.float32),
            pltpu.VMEM_SHARED((N_ACC, K), jnp.float32),
            pltpu.SemaphoreType.DMA,
            pltpu.SemaphoreType.DMA,
            pltpu.SemaphoreType.DMA,
            pltpu.SemaphoreType.DMA,
        ],
    )
    def k(u_hbm, v_hbm, ei_hbm, ea_hbm, wd_hbm, out_hbm,
          srcg, dstg, eag, ub, vb, sb, wdb, acc, sem_ig, sem_ga0, sem_ga1,
          sem_sc):
        sem_ga = (sem_ga0, sem_ga1)
        cid = lax.axis_index("c")
        sid = lax.axis_index("s")
        wid = cid * NUM_SUBCORES + sid

        def zrow(i, _):
            ub[0, i, :] = jnp.zeros((K,), jnp.float32)
            return 0
        lax.fori_loop(0, GCH, zrow, 0)
        zbase = sid * ZROWS

        def zchunk(j, _):
            pltpu.sync_copy(ub.at[0], acc.at[pl.ds(zbase + j * GCH, GCH)])
            return 0
        lax.fori_loop(0, ZROWS // GCH, zchunk, 0)
        pltpu.sync_copy(ub.at[0, pl.ds(0, ZROWS % GCH)],
                        acc.at[pl.ds(zbase + (ZROWS // GCH) * GCH, ZROWS % GCH)])
        plsc.subcore_barrier()

        pltpu.sync_copy(wd_hbm, wdb)
        wd2v = wdb[:]
        one = jnp.float32(1.0)
        two = jnp.float32(2.0)
        lanes = [jnp.full((K, 1), i, jnp.int32) for i in range(K)]
        _dnums = lax.GatherDimensionNumbers(
            offset_dims=(), collapsed_slice_dims=(0,), start_index_map=(0,))

        def _bcast(av, i):
            return lax.gather(av, lanes[i], _dnums, (1,),
                              mode=lax.GatherScatterMode.PROMISE_IN_BOUNDS)

        def idx_issue(g):
            b = wid * EPT + g * GCH
            s = lax.rem(g, 3)
            pltpu.async_copy(ei_hbm.at[0, pl.ds(b, GCH)], srcg.at[s], sem_ig)
            pltpu.async_copy(ei_hbm.at[1, pl.ds(b, GCH)], dstg.at[s], sem_ig)
            pltpu.async_copy(ea_hbm.at[pl.ds(b, GCH)], eag.at[s], sem_ig)

        def idx_wait(g):
            b = wid * EPT + g * GCH
            s = lax.rem(g, 3)
            pltpu.make_async_copy(
                ei_hbm.at[0, pl.ds(b, GCH)], srcg.at[s], sem_ig).wait()
            pltpu.make_async_copy(
                ei_hbm.at[1, pl.ds(b, GCH)], dstg.at[s], sem_ig).wait()
            pltpu.make_async_copy(
                ea_hbm.at[pl.ds(b, GCH)], eag.at[s], sem_ig).wait()

        def gath_issue(g, p):
            s = lax.rem(g, 3)
            pltpu.async_copy(u_hbm.at[srcg.at[s]], ub.at[p], sem_ga[p])
            pltpu.async_copy(v_hbm.at[dstg.at[s]], vb.at[p], sem_ga[p])

        def gath_wait(g, p):
            s = lax.rem(g, 3)
            pltpu.make_async_copy(
                u_hbm.at[srcg.at[s]], ub.at[p], sem_ga[p]).wait()
            pltpu.make_async_copy(
                v_hbm.at[dstg.at[s]], vb.at[p], sem_ga[p]).wait()

        def scat_issue(g, p):
            s = lax.rem(g, 3)
            pltpu.async_copy(sb.at[p], acc.at[dstg.at[s]], sem_sc, add=True)

        def scat_wait(g, p):
            s = lax.rem(g, 3)
            pltpu.make_async_copy(sb.at[p], acc.at[dstg.at[s]], sem_sc).wait()

        def edge16(p, av, base_e, lo):
            for i in range(lo, K):
                e = base_e + i
                ai = _bcast(av, i)
                z = ub[p, e, :] + vb[p, e, :] + ai * wd2v
                sb[p, e, :] = one - two / (jnp.exp(z) + one)

        def compute(g, p):
            s = lax.rem(g, 3)

            def qblk(q, _):
                av = eag[s, pl.ds(q * K, K)]
                edge16(p, av, q * K, 0)
                return 0
            lax.fori_loop(0, GCH // K, qblk, 0)
            av = eag[s, pl.ds(GCH - K, K)]
            edge16(p, av, GCH - K, K - (GCH - (GCH // K) * K))

        def step(g, p):
            @pl.when(g > 0)
            def _():
                scat_wait(g - 1, 1 - p)

            @pl.when(g < GPT - 1)
            def _():
                idx_wait(g + 1)

            @pl.when(g < GPT - 2)
            def _():
                idx_issue(g + 2)

            @pl.when(g < GPT - 1)
            def _():
                gath_issue(g + 1, 1 - p)
            gath_wait(g, p)
            compute(g, p)
            scat_issue(g, p)

        idx_issue(0)
        idx_issue(1)
        idx_wait(0)
        gath_issue(0, 0)

        def body(t, _):
            step(2 * t, 0)
            step(2 * t + 1, 1)
            return 0
        lax.fori_loop(0, GPT // 2, body, 0)
        scat_wait(GPT - 1, 1)
        plsc.subcore_barrier()

        @pl.when(sid == 0)
        def _():
            pltpu.sync_copy(acc, out_hbm.at[cid])

    return k(U, V, ei, eaf, wd2)


def _pool(parts, batchp):
    @functools.partial(
        pl.kernel,
        mesh=_mesh(),
        compiler_params=pltpu.CompilerParams(use_tc_tiling_on_sc=False),
        out_type=(
            jax.ShapeDtypeStruct((NUM_CORES, G, K), jnp.float32),
            jax.ShapeDtypeStruct((NUM_CORES, G, K), jnp.float32),
        ),
        scratch_types=[
            pltpu.VMEM((PCH,), jnp.int32),
            pltpu.VMEM((PCH, K), jnp.float32),
            pltpu.VMEM((PCH, K), jnp.float32),
            pltpu.VMEM((PCH, K), jnp.float32),
            pltpu.VMEM_SHARED((G_ACC, K), jnp.float32),
            pltpu.VMEM_SHARED((G_ACC, K), jnp.float32),
        ],
    )
    def k(parts_hbm, batch_hbm, rsum_hbm, cnt_hbm,
          bb, h0, h1, onesb, rs, cs):
        cid = lax.axis_index("c")
        sid = lax.axis_index("s")
        wid = cid * NUM_SUBCORES + sid

        def fill(i, _):
            h0[i, :] = jnp.zeros((K,), jnp.float32)
            onesb[i, :] = jnp.ones((K,), jnp.float32)
            return 0
        lax.fori_loop(0, PCH, fill, 0)
        zb = sid * GZ
        pltpu.sync_copy(h0.at[pl.ds(0, GZ)], rs.at[pl.ds(zb, GZ)])
        pltpu.sync_copy(h0.at[pl.ds(0, GZ)], cs.at[pl.ds(zb, GZ)])
        plsc.subcore_barrier()

        def chunk(ch, _):
            g = wid * PCT + ch
            pltpu.sync_copy(batch_hbm.at[g], bb)
            pltpu.sync_copy(parts_hbm.at[0, pl.ds(g * PCH, PCH)], h0)
            pltpu.sync_copy(parts_hbm.at[1, pl.ds(g * PCH, PCH)], h1)
            pltpu.sync_copy(h0, rs.at[bb], add=True)
            pltpu.sync_copy(h1, rs.at[bb], add=True)
            pltpu.sync_copy(onesb, cs.at[bb], add=True)
            return 0
        lax.fori_loop(0, PCT, chunk, 0)
        plsc.subcore_barrier()

        @pl.when(sid == 0)
        def _():
            pltpu.sync_copy(rs, rsum_hbm.at[cid])
            pltpu.sync_copy(cs, cnt_hbm.at[cid])

    return k(parts, batchp)


_R = 5000


def _proj0(xp, Wu, Wv, bv):
    def body(xp_ref, wu_ref, wv_ref, bv_ref, u_ref, v_ref):
        xpb = xp_ref[...]
        u_ref[...] = jnp.dot(xpb, wu_ref[...], preferred_element_type=jnp.float32)
        v_ref[...] = (jnp.dot(xpb, wv_ref[...], preferred_element_type=jnp.float32)
                      + bv_ref[...])

    return pl.pallas_call(
        body,
        grid=(N // _R,),
        in_specs=[
            pl.BlockSpec((_R, 8), lambda i: (i, 0)),
            pl.BlockSpec((8, K), lambda i: (0, 0)),
            pl.BlockSpec((8, K), lambda i: (0, 0)),
            pl.BlockSpec((1, K), lambda i: (0, 0)),
        ],
        out_specs=[
            pl.BlockSpec((_R, K), lambda i: (i, 0)),
            pl.BlockSpec((_R, K), lambda i: (i, 0)),
        ],
        out_shape=[
            jax.ShapeDtypeStruct((N, K), jnp.float32),
            jax.ShapeDtypeStruct((N, K), jnp.float32),
        ],
    )(xp, Wu, Wv, bv)


def _proj1(parts, p, Wua, Wuc, Wva, Wvc, bv):
    def body(pa_ref, pb_ref, p_ref, wua_ref, wuc_ref, wva_ref, wvc_ref,
             bv_ref, u_ref, v_ref):
        h = pa_ref[0] + pb_ref[0]
        pb = p_ref[...]
        u_ref[...] = (jnp.dot(h, wua_ref[...], preferred_element_type=jnp.float32)
                      + jnp.dot(pb, wuc_ref[...], preferred_element_type=jnp.float32))
        v_ref[...] = (jnp.dot(h, wva_ref[...], preferred_element_type=jnp.float32)
                      + jnp.dot(pb, wvc_ref[...], preferred_element_type=jnp.float32)
                      + bv_ref[...])

    return pl.pallas_call(
        body,
        grid=(N // _R,),
        in_specs=[
            pl.BlockSpec((1, _R, K), lambda i: (0, i, 0)),
            pl.BlockSpec((1, _R, K), lambda i: (1, i, 0)),
            pl.BlockSpec((_R, 3), lambda i: (i, 0)),
            pl.BlockSpec((K, K), lambda i: (0, 0)),
            pl.BlockSpec((3, K), lambda i: (0, 0)),
            pl.BlockSpec((K, K), lambda i: (0, 0)),
            pl.BlockSpec((3, K), lambda i: (0, 0)),
            pl.BlockSpec((1, K), lambda i: (0, 0)),
        ],
        out_specs=[
            pl.BlockSpec((_R, K), lambda i: (i, 0)),
            pl.BlockSpec((_R, K), lambda i: (i, 0)),
        ],
        out_shape=[
            jax.ShapeDtypeStruct((N, K), jnp.float32),
            jax.ShapeDtypeStruct((N, K), jnp.float32),
        ],
    )(parts, parts, p, Wua, Wuc, Wva, Wvc, bv)


def _final(rsum, cnt, Wp, bp):
    def body(rs_ref, cn_ref, wp_ref, bp_ref, pred_ref, rep_ref):
        rs = rs_ref[0] + rs_ref[1]
        c = cn_ref[0, :, 0:1] + cn_ref[1, :, 0:1]
        rep = rs / jnp.maximum(c, 1.0)
        rep_ref[...] = rep
        pred_ref[...] = (jnp.dot(rep, wp_ref[...], preferred_element_type=jnp.float32)
                         + bp_ref[...])

    return pl.pallas_call(
        body,
        out_shape=[
            jax.ShapeDtypeStruct((G, 1), jnp.float32),
            jax.ShapeDtypeStruct((G, K), jnp.float32),
        ],
    )(rsum, cnt, Wp, bp)


def kernel(x, p, edge_index, edge_attr, batch, W0, b0, W1, b1, Wp, bp):
    eaf = edge_attr.reshape(E)
    batchp = batch.reshape(-1, PCH)

    Wa0, Wb0, Wc0, wd0 = W0[0:5], W0[5:10], W0[10:13], W0[13]
    Wu0 = 2.0 * jnp.concatenate([Wa0, -Wc0], axis=0)
    Wv0 = 2.0 * jnp.concatenate([Wb0, Wc0], axis=0)
    bv0 = (2.0 * b0).reshape(1, K)
    wd20 = 2.0 * wd0
    xp = jnp.concatenate([x, p], axis=1)

    U0, V0 = _proj0(xp, Wu0, Wv0, bv0)
    parts0 = _edge_pass(U0, V0, edge_index, eaf, wd20)

    Wa1, Wb1, Wc1, wd1 = W1[0:16], W1[16:32], W1[32:35], W1[35]
    U1, V1 = _proj1(parts0, p, 2.0 * Wa1, -2.0 * Wc1, 2.0 * Wb1, 2.0 * Wc1,
                    (2.0 * b1).reshape(1, K))
    parts1 = _edge_pass(U1, V1, edge_index, eaf, 2.0 * wd1)

    rsum, cnt = _pool(parts1, batchp)
    pred, rep = _final(rsum, cnt, Wp, bp.reshape(1, 1))
    return (pred, rep)
